# Initial kernel scaffold; baseline (speedup 1.0000x reference)
import jax
import jax.numpy as jnp
from jax.experimental import pallas as pl


def kernel(input, params):
    raise NotImplementedError("write your pallas kernel here")


# speedup vs baseline: 1.0425x; 1.0425x over previous
import functools

import jax
import jax.numpy as jnp
from jax import lax
from jax.experimental import pallas as pl
from jax.experimental.pallas import tpu as pltpu

_EPS = 1e-5
_NEG = -1e30

_SA_CFGS = [
    dict(npoint=2048, radius=0.2, nsample=64),
    dict(npoint=1024, radius=0.4, nsample=32),
    dict(npoint=512, radius=0.8, nsample=16),
    dict(npoint=256, radius=1.2, nsample=16),
]

_INTERPRET = False


def _bn_relu(x, m, sq, g, bt, fl):
    return jnp.maximum((x - m) / sq * g + bt, fl)


def _mlp_stage_body(x_ref, m_ref, sq_ref, g_ref, bt_ref, fl_ref,
                    w_ref, b_ref, out_ref, pool_ref=None, *, s_blk, K, pool):
    Cin = x_ref.shape[-1]
    Cout = w_ref.shape[0]
    x = x_ref[0].reshape(s_blk * K, Cin)
    t = _bn_relu(x, m_ref[0][None, :], sq_ref[0][None, :], g_ref[0][None, :],
                 bt_ref[0][None, :], fl_ref[0][None, :])
    z = lax.dot_general(t, w_ref[...], (((1,), (1,)), ((), ())),
                        preferred_element_type=jnp.float32)
    z = z + b_ref[0][None, :]
    z3 = z.reshape(s_blk, K, Cout)
    out_ref[0] = z3
    if pool:
        pool_ref[0] = jnp.max(z3, axis=1)


def _mlp_stage(x_raw, m, sq, g, bt, fl, W, b, *, pool, s_blk=128):
    B, S, K, Cin = x_raw.shape
    Cout = W.shape[0]
    grid = (B, S // s_blk)
    out_shape = [jax.ShapeDtypeStruct((B, S, K, Cout), jnp.float32)]
    out_specs = [pl.BlockSpec((1, s_blk, K, Cout), lambda bi, si: (bi, si, 0, 0))]
    if pool:
        out_shape.append(jax.ShapeDtypeStruct((B, S, Cout), jnp.float32))
        out_specs.append(pl.BlockSpec((1, s_blk, Cout), lambda bi, si: (bi, si, 0)))
    vec = lambda: pl.BlockSpec((1, Cin), lambda bi, si: (0, 0))
    res = pl.pallas_call(
        functools.partial(_mlp_stage_body, s_blk=s_blk, K=K, pool=pool),
        grid=grid,
        in_specs=[
            pl.BlockSpec((1, s_blk, K, Cin), lambda bi, si: (bi, si, 0, 0)),
            vec(), vec(), vec(), vec(), vec(),
            pl.BlockSpec((Cout, Cin), lambda bi, si: (0, 0)),
            pl.BlockSpec((1, Cout), lambda bi, si: (0, 0)),
        ],
        out_specs=out_specs,
        out_shape=out_shape,
        interpret=_INTERPRET,
    )(x_raw, m[None, :], sq[None, :], g[None, :], bt[None, :], fl[None, :],
      W, b[None, :])
    if pool:
        return res[0], res[1]
    return res[0], None


def _stats(z):
    mean = jnp.mean(z, axis=(0, 1, 2))
    var = jnp.var(z, axis=(0, 1, 2))
    return mean, jnp.sqrt(var + _EPS)


def _epilogue_body(x_ref, m_ref, sq_ref, g_ref, bt_ref, o_ref):
    o_ref[...] = jnp.maximum(
        (x_ref[...] - m_ref[0][None, None, :]) / sq_ref[0][None, None, :]
        * g_ref[0][None, None, :] + bt_ref[0][None, None, :], 0.0)


def _epilogue(pooled_raw, m, sq, g, bt):
    B, S, C = pooled_raw.shape
    return pl.pallas_call(
        _epilogue_body,
        out_shape=jax.ShapeDtypeStruct((B, S, C), jnp.float32),
        interpret=_INTERPRET,
    )(pooled_raw, m[None, :], sq[None, :], g[None, :], bt[None, :])


def _sq_dist(src, dst):
    return (jnp.sum(src ** 2, -1)[:, :, None]
            + jnp.sum(dst ** 2, -1)[:, None, :]
            - 2.0 * jnp.einsum('bsc,bnc->bsn', src, dst))


def _fps(xyz, npoint):
    B, N, _ = xyz.shape

    def body(i, state):
        centroids, distance, farthest = state
        centroids = centroids.at[:, i].set(farthest)
        centroid = jnp.take_along_axis(xyz, farthest[:, None, None], axis=1)
        dist = jnp.sum((xyz - centroid) ** 2, -1)
        distance = jnp.minimum(distance, dist)
        farthest = jnp.argmax(distance, -1).astype(jnp.int32)
        return centroids, distance, farthest

    centroids = jnp.zeros((B, npoint), dtype=jnp.int32)
    distance = jnp.full((B, N), 1e10, dtype=xyz.dtype)
    farthest = jnp.zeros((B,), dtype=jnp.int32)
    centroids, _, _ = lax.fori_loop(0, npoint, body,
                                    (centroids, distance, farthest))
    return centroids


def _query_ball(radius, nsample, xyz, new_xyz):
    B, N, _ = xyz.shape
    S = new_xyz.shape[1]
    sqrdists = _sq_dist(new_xyz, xyz)
    group_idx = jnp.broadcast_to(jnp.arange(N, dtype=jnp.int32), (B, S, N))
    group_idx = jnp.where(sqrdists > radius ** 2, N, group_idx)
    group_idx = jnp.sort(group_idx, axis=-1)[:, :, :nsample]
    group_first = jnp.broadcast_to(group_idx[:, :, :1], group_idx.shape)
    return jnp.where(group_idx == N, group_first, group_idx)


def _gather_rows(points, idx):
    B = points.shape[0]
    batch = jnp.arange(B).reshape((B,) + (1,) * (idx.ndim - 1))
    return points[batch, idx]


def _sa_layer(cfg, lps, xyz, feats_raw, fstat):
    B, N, _ = xyz.shape
    S, K = cfg['npoint'], cfg['nsample']
    fps_idx = _fps(xyz, S)
    new_xyz = jnp.take_along_axis(xyz, fps_idx[..., None], axis=1)
    idx = _query_ball(cfg['radius'], K, xyz, new_xyz)
    gx = _gather_rows(xyz, idx) - new_xyz[:, :, None, :]
    one3 = jnp.ones((3,), jnp.float32)
    zero3 = jnp.zeros((3,), jnp.float32)
    neg3 = jnp.full((3,), _NEG, jnp.float32)
    if feats_raw is None:
        X = gx
        m, sq, g, bt, fl = zero3, one3, one3, zero3, neg3
    else:
        gf = _gather_rows(feats_raw, idx)
        X = jnp.concatenate([gx, gf], axis=-1)
        fm, fsq, fg, fb = fstat
        m = jnp.concatenate([zero3, fm])
        sq = jnp.concatenate([one3, fsq])
        g = jnp.concatenate([one3, fg])
        bt = jnp.concatenate([zero3, fb])
        fl = jnp.concatenate([neg3, jnp.zeros_like(fm)])
    pooled = None
    for i, (W, b, gamma, beta) in enumerate(lps):
        pool = (i == len(lps) - 1)
        z, pooled = _mlp_stage(X, m, sq, g, bt, fl, W, b, pool=pool)
        zm, zsq = _stats(z)
        m, sq, g, bt = zm, zsq, gamma, beta
        fl = jnp.zeros_like(zm)
        X = z
    return new_xyz, pooled, (m, sq, g, bt)


def kernel(input, params):
    xyz = jnp.transpose(input, (0, 2, 1))
    feats, fstat = None, None
    for cfg, lps in zip(_SA_CFGS, params):
        xyz, feats, fstat = _sa_layer(cfg, lps, xyz, feats, fstat)
    f = _epilogue(feats, *fstat)
    return (jnp.transpose(xyz, (0, 2, 1)), jnp.transpose(f, (0, 2, 1)))


# speedup vs baseline: 2.1382x; 2.0510x over previous
import functools

import jax
import jax.numpy as jnp
from jax import lax
from jax.experimental import pallas as pl
from jax.experimental.pallas import tpu as pltpu

_EPS = 1e-5
_NEG = -1e30

_SA_CFGS = [
    dict(npoint=2048, radius=0.2, nsample=64),
    dict(npoint=1024, radius=0.4, nsample=32),
    dict(npoint=512, radius=0.8, nsample=16),
    dict(npoint=256, radius=1.2, nsample=16),
]

_INTERPRET = False


def _bn_relu(x, m, sq, g, bt, fl):
    return jnp.maximum((x - m) / sq * g + bt, fl)


def _mlp_stage_body(x_ref, m_ref, sq_ref, g_ref, bt_ref, fl_ref,
                    w_ref, b_ref, out_ref, pool_ref=None, *, s_blk, K, pool):
    Cin = x_ref.shape[-1]
    Cout = w_ref.shape[0]
    x = x_ref[0].reshape(s_blk * K, Cin)
    t = _bn_relu(x, m_ref[0][None, :], sq_ref[0][None, :], g_ref[0][None, :],
                 bt_ref[0][None, :], fl_ref[0][None, :])
    z = lax.dot_general(t, w_ref[...], (((1,), (1,)), ((), ())),
                        preferred_element_type=jnp.float32)
    z = z + b_ref[0][None, :]
    z3 = z.reshape(s_blk, K, Cout)
    out_ref[0] = z3
    if pool:
        pool_ref[0] = jnp.max(z3, axis=1)


def _mlp_stage(x_raw, m, sq, g, bt, fl, W, b, *, pool, s_blk=128):
    B, S, K, Cin = x_raw.shape
    Cout = W.shape[0]
    grid = (B, S // s_blk)
    out_shape = [jax.ShapeDtypeStruct((B, S, K, Cout), jnp.float32)]
    out_specs = [pl.BlockSpec((1, s_blk, K, Cout), lambda bi, si: (bi, si, 0, 0))]
    if pool:
        out_shape.append(jax.ShapeDtypeStruct((B, S, Cout), jnp.float32))
        out_specs.append(pl.BlockSpec((1, s_blk, Cout), lambda bi, si: (bi, si, 0)))
    vec = lambda: pl.BlockSpec((1, Cin), lambda bi, si: (0, 0))
    res = pl.pallas_call(
        functools.partial(_mlp_stage_body, s_blk=s_blk, K=K, pool=pool),
        grid=grid,
        in_specs=[
            pl.BlockSpec((1, s_blk, K, Cin), lambda bi, si: (bi, si, 0, 0)),
            vec(), vec(), vec(), vec(), vec(),
            pl.BlockSpec((Cout, Cin), lambda bi, si: (0, 0)),
            pl.BlockSpec((1, Cout), lambda bi, si: (0, 0)),
        ],
        out_specs=out_specs,
        out_shape=out_shape,
        interpret=_INTERPRET,
    )(x_raw, m[None, :], sq[None, :], g[None, :], bt[None, :], fl[None, :],
      W, b[None, :])
    if pool:
        return res[0], res[1]
    return res[0], None


def _stats(z):
    mean = jnp.mean(z, axis=(0, 1, 2))
    var = jnp.var(z, axis=(0, 1, 2))
    return mean, jnp.sqrt(var + _EPS)


def _epilogue_body(x_ref, m_ref, sq_ref, g_ref, bt_ref, o_ref):
    o_ref[...] = jnp.maximum(
        (x_ref[...] - m_ref[0][None, None, :]) / sq_ref[0][None, None, :]
        * g_ref[0][None, None, :] + bt_ref[0][None, None, :], 0.0)


def _epilogue(pooled_raw, m, sq, g, bt):
    B, S, C = pooled_raw.shape
    return pl.pallas_call(
        _epilogue_body,
        out_shape=jax.ShapeDtypeStruct((B, S, C), jnp.float32),
        interpret=_INTERPRET,
    )(pooled_raw, m[None, :], sq[None, :], g[None, :], bt[None, :])


def _fps_body(x_ref, y_ref, z_ref, o_ref, *, npoint, N, B):
    NB = x_ref.shape[1]
    x = x_ref[...]
    y = y_ref[...]
    z = z_ref[...]
    sub = lax.broadcasted_iota(jnp.int32, (B, NB, 128), 1)
    lane = lax.broadcasted_iota(jnp.int32, (B, NB, 128), 2)
    flat = sub * 128 + lane
    out_lane = lax.broadcasted_iota(jnp.int32, (1, 128), 1)

    def body(i, carry):
        dist, f = carry
        msk = (flat == f)
        cx = jnp.sum(jnp.where(msk, x, 0.0), axis=(1, 2), keepdims=True)
        cy = jnp.sum(jnp.where(msk, y, 0.0), axis=(1, 2), keepdims=True)
        cz = jnp.sum(jnp.where(msk, z, 0.0), axis=(1, 2), keepdims=True)
        row = jnp.zeros((1, 128), jnp.float32)
        for b in range(B):
            row = jnp.where(out_lane == 3 * b + 0, cx[b, 0, 0], row)
            row = jnp.where(out_lane == 3 * b + 1, cy[b, 0, 0], row)
            row = jnp.where(out_lane == 3 * b + 2, cz[b, 0, 0], row)
        o_ref[pl.ds(i, 1), :] = row
        dx = x - cx
        dy = y - cy
        dz = z - cz
        d = (dx * dx + dy * dy) + dz * dz
        dist = jnp.minimum(dist, d)
        m = jnp.max(dist, axis=(1, 2), keepdims=True)
        f = jnp.min(jnp.where(dist == m, flat, N), axis=(1, 2), keepdims=True)
        return dist, f

    dist0 = jnp.full((B, NB, 128), 1e10, jnp.float32)
    f0 = jnp.zeros((B, 1, 1), jnp.int32)
    lax.fori_loop(0, npoint, body, (dist0, f0))


def _fps_pallas(xyz, npoint):
    B, N, _ = xyz.shape
    NB = N // 128
    coords = jnp.transpose(xyz, (0, 2, 1)).reshape(B, 3, NB, 128)
    out = pl.pallas_call(
        functools.partial(_fps_body, npoint=npoint, N=N, B=B),
        out_shape=jax.ShapeDtypeStruct((npoint, 128), jnp.float32),
        interpret=_INTERPRET,
    )(coords[:, 0], coords[:, 1], coords[:, 2])
    return jnp.transpose(out[:, :3 * B].reshape(npoint, B, 3), (1, 0, 2))


def _sq_dist(src, dst):
    return (jnp.sum(src ** 2, -1)[:, :, None]
            + jnp.sum(dst ** 2, -1)[:, None, :]
            - 2.0 * jnp.einsum('bsc,bnc->bsn', src, dst))


def _fps(xyz, npoint):
    B, N, _ = xyz.shape

    def body(i, state):
        centroids, distance, farthest = state
        centroids = centroids.at[:, i].set(farthest)
        centroid = jnp.take_along_axis(xyz, farthest[:, None, None], axis=1)
        dist = jnp.sum((xyz - centroid) ** 2, -1)
        distance = jnp.minimum(distance, dist)
        farthest = jnp.argmax(distance, -1).astype(jnp.int32)
        return centroids, distance, farthest

    centroids = jnp.zeros((B, npoint), dtype=jnp.int32)
    distance = jnp.full((B, N), 1e10, dtype=xyz.dtype)
    farthest = jnp.zeros((B,), dtype=jnp.int32)
    centroids, _, _ = lax.fori_loop(0, npoint, body,
                                    (centroids, distance, farthest))
    return centroids


def _query_ball(radius, nsample, xyz, new_xyz):
    B, N, _ = xyz.shape
    S = new_xyz.shape[1]
    sqrdists = _sq_dist(new_xyz, xyz)
    group_idx = jnp.broadcast_to(jnp.arange(N, dtype=jnp.int32), (B, S, N))
    group_idx = jnp.where(sqrdists > radius ** 2, N, group_idx)
    group_idx = jnp.sort(group_idx, axis=-1)[:, :, :nsample]
    group_first = jnp.broadcast_to(group_idx[:, :, :1], group_idx.shape)
    return jnp.where(group_idx == N, group_first, group_idx)


def _gather_rows(points, idx):
    B = points.shape[0]
    batch = jnp.arange(B).reshape((B,) + (1,) * (idx.ndim - 1))
    return points[batch, idx]


def _sa_layer(cfg, lps, xyz, feats_raw, fstat):
    B, N, _ = xyz.shape
    S, K = cfg['npoint'], cfg['nsample']
    new_xyz = _fps_pallas(xyz, S)
    idx = _query_ball(cfg['radius'], K, xyz, new_xyz)
    gx = _gather_rows(xyz, idx) - new_xyz[:, :, None, :]
    one3 = jnp.ones((3,), jnp.float32)
    zero3 = jnp.zeros((3,), jnp.float32)
    neg3 = jnp.full((3,), _NEG, jnp.float32)
    if feats_raw is None:
        X = gx
        m, sq, g, bt, fl = zero3, one3, one3, zero3, neg3
    else:
        gf = _gather_rows(feats_raw, idx)
        X = jnp.concatenate([gx, gf], axis=-1)
        fm, fsq, fg, fb = fstat
        m = jnp.concatenate([zero3, fm])
        sq = jnp.concatenate([one3, fsq])
        g = jnp.concatenate([one3, fg])
        bt = jnp.concatenate([zero3, fb])
        fl = jnp.concatenate([neg3, jnp.zeros_like(fm)])
    pooled = None
    for i, (W, b, gamma, beta) in enumerate(lps):
        pool = (i == len(lps) - 1)
        z, pooled = _mlp_stage(X, m, sq, g, bt, fl, W, b, pool=pool)
        zm, zsq = _stats(z)
        m, sq, g, bt = zm, zsq, gamma, beta
        fl = jnp.zeros_like(zm)
        X = z
    return new_xyz, pooled, (m, sq, g, bt)


def kernel(input, params):
    xyz = jnp.transpose(input, (0, 2, 1))
    feats, fstat = None, None
    for cfg, lps in zip(_SA_CFGS, params):
        xyz, feats, fstat = _sa_layer(cfg, lps, xyz, feats, fstat)
    f = _epilogue(feats, *fstat)
    return (jnp.transpose(xyz, (0, 2, 1)), jnp.transpose(f, (0, 2, 1)))


# speedup vs baseline: 2.1397x; 1.0007x over previous
import functools

import jax
import jax.numpy as jnp
import numpy as np
from jax import lax
from jax.experimental import pallas as pl
from jax.experimental.pallas import tpu as pltpu
from jax.experimental.pallas import tpu_sc as plsc

_EPS = 1e-5
_NEG = -1e30

_SA_CFGS = [
    dict(npoint=2048, radius=0.2, nsample=64),
    dict(npoint=1024, radius=0.4, nsample=32),
    dict(npoint=512, radius=0.8, nsample=16),
    dict(npoint=256, radius=1.2, nsample=16),
]

_INTERPRET = False


def _bn_relu(x, m, sq, g, bt, fl):
    return jnp.maximum((x - m) / sq * g + bt, fl)


def _mlp_stage_body(x_ref, m_ref, sq_ref, g_ref, bt_ref, fl_ref,
                    w_ref, b_ref, out_ref, pool_ref=None, *, s_blk, K, pool):
    Cin = x_ref.shape[-1]
    Cout = w_ref.shape[0]
    x = x_ref[0].reshape(s_blk * K, Cin)
    t = _bn_relu(x, m_ref[0][None, :], sq_ref[0][None, :], g_ref[0][None, :],
                 bt_ref[0][None, :], fl_ref[0][None, :])
    z = lax.dot_general(t, w_ref[...], (((1,), (1,)), ((), ())),
                        preferred_element_type=jnp.float32)
    z = z + b_ref[0][None, :]
    z3 = z.reshape(s_blk, K, Cout)
    out_ref[0] = z3
    if pool:
        pool_ref[0] = jnp.max(z3, axis=1)


def _mlp_stage(x_raw, m, sq, g, bt, fl, W, b, *, pool, s_blk=128):
    B, S, K, Cin = x_raw.shape
    Cout = W.shape[0]
    grid = (B, S // s_blk)
    out_shape = [jax.ShapeDtypeStruct((B, S, K, Cout), jnp.float32)]
    out_specs = [pl.BlockSpec((1, s_blk, K, Cout), lambda bi, si: (bi, si, 0, 0))]
    if pool:
        out_shape.append(jax.ShapeDtypeStruct((B, S, Cout), jnp.float32))
        out_specs.append(pl.BlockSpec((1, s_blk, Cout), lambda bi, si: (bi, si, 0)))
    vec = lambda: pl.BlockSpec((1, Cin), lambda bi, si: (0, 0))
    res = pl.pallas_call(
        functools.partial(_mlp_stage_body, s_blk=s_blk, K=K, pool=pool),
        grid=grid,
        in_specs=[
            pl.BlockSpec((1, s_blk, K, Cin), lambda bi, si: (bi, si, 0, 0)),
            vec(), vec(), vec(), vec(), vec(),
            pl.BlockSpec((Cout, Cin), lambda bi, si: (0, 0)),
            pl.BlockSpec((1, Cout), lambda bi, si: (0, 0)),
        ],
        out_specs=out_specs,
        out_shape=out_shape,
        interpret=_INTERPRET,
    )(x_raw, m[None, :], sq[None, :], g[None, :], bt[None, :], fl[None, :],
      W, b[None, :])
    if pool:
        return res[0], res[1]
    return res[0], None


def _stats(z):
    mean = jnp.mean(z, axis=(0, 1, 2))
    var = jnp.var(z, axis=(0, 1, 2))
    return mean, jnp.sqrt(var + _EPS)


def _epilogue_body(x_ref, m_ref, sq_ref, g_ref, bt_ref, o_ref):
    o_ref[...] = jnp.maximum(
        (x_ref[...] - m_ref[0][None, None, :]) / sq_ref[0][None, None, :]
        * g_ref[0][None, None, :] + bt_ref[0][None, None, :], 0.0)


def _epilogue(pooled_raw, m, sq, g, bt):
    B, S, C = pooled_raw.shape
    return pl.pallas_call(
        _epilogue_body,
        out_shape=jax.ShapeDtypeStruct((B, S, C), jnp.float32),
        interpret=_INTERPRET,
    )(pooled_raw, m[None, :], sq[None, :], g[None, :], bt[None, :])


def _fps_body(x_ref, y_ref, z_ref, o_ref, *, npoint, N, B):
    NB = x_ref.shape[1]
    x = x_ref[...]
    y = y_ref[...]
    z = z_ref[...]
    sub = lax.broadcasted_iota(jnp.int32, (B, NB, 128), 1)
    lane = lax.broadcasted_iota(jnp.int32, (B, NB, 128), 2)
    flat = sub * 128 + lane
    out_lane = lax.broadcasted_iota(jnp.int32, (1, 128), 1)

    def body(i, carry):
        dist, f = carry
        msk = (flat == f)
        cx = jnp.sum(jnp.where(msk, x, 0.0), axis=(1, 2), keepdims=True)
        cy = jnp.sum(jnp.where(msk, y, 0.0), axis=(1, 2), keepdims=True)
        cz = jnp.sum(jnp.where(msk, z, 0.0), axis=(1, 2), keepdims=True)
        row = jnp.zeros((1, 128), jnp.float32)
        for b in range(B):
            row = jnp.where(out_lane == 3 * b + 0, cx[b, 0, 0], row)
            row = jnp.where(out_lane == 3 * b + 1, cy[b, 0, 0], row)
            row = jnp.where(out_lane == 3 * b + 2, cz[b, 0, 0], row)
        o_ref[pl.ds(i, 1), :] = row
        dx = x - cx
        dy = y - cy
        dz = z - cz
        d = (dx * dx + dy * dy) + dz * dz
        dist = jnp.minimum(dist, d)
        m = jnp.max(dist, axis=(1, 2), keepdims=True)
        f = jnp.min(jnp.where(dist == m, flat, N), axis=(1, 2), keepdims=True)
        return dist, f

    dist0 = jnp.full((B, NB, 128), 1e10, jnp.float32)
    f0 = jnp.zeros((B, 1, 1), jnp.int32)
    lax.fori_loop(0, npoint, body, (dist0, f0))


def _fps_pallas(xyz, npoint):
    B, N, _ = xyz.shape
    NB = N // 128
    coords = jnp.transpose(xyz, (0, 2, 1)).reshape(B, 3, NB, 128)
    out = pl.pallas_call(
        functools.partial(_fps_body, npoint=npoint, N=N, B=B),
        out_shape=jax.ShapeDtypeStruct((npoint, 128), jnp.float32),
        interpret=_INTERPRET,
    )(coords[:, 0], coords[:, 1], coords[:, 2])
    return jnp.transpose(out[:, :3 * B].reshape(npoint, B, 3), (1, 0, 2))


def _ballq_words_body(nx_ref, xyz_ref, o_ref, *, r2, s_blk, N):
    nx = nx_ref[0]
    xyzb = xyz_ref[0]
    ns = jnp.sum(nx * nx, axis=0, keepdims=True)
    nn = jnp.sum(xyzb * xyzb, axis=1, keepdims=True)
    dot = lax.dot_general(xyzb, nx, (((1,), (0,)), ((), ())),
                          preferred_element_type=jnp.float32)
    d = (ns + nn) - 2.0 * dot
    valid = jnp.where(d > r2, 0.0, 1.0)
    W = N // 16
    v3 = valid.reshape(W, 16, s_blk)
    pw = jnp.left_shift(1, lax.broadcasted_iota(jnp.int32, (1, 16, 1), 1))
    words = jnp.sum(v3 * pw.astype(jnp.float32), axis=1)
    o_ref[0] = words.astype(jnp.int32)


def _ballq_words(xyz, new_xyz, radius, *, s_blk=128):
    B, N, _ = xyz.shape
    S = new_xyz.shape[1]
    W = N // 16
    nxT = jnp.transpose(new_xyz, (0, 2, 1))
    out = pl.pallas_call(
        functools.partial(_ballq_words_body, r2=float(radius ** 2),
                          s_blk=s_blk, N=N),
        grid=(B, S // s_blk),
        in_specs=[
            pl.BlockSpec((1, 3, s_blk), lambda bi, si: (bi, 0, si)),
            pl.BlockSpec((1, N, 3), lambda bi, si: (bi, 0, 0)),
        ],
        out_specs=pl.BlockSpec((1, W, s_blk), lambda bi, si: (bi, 0, si)),
        out_shape=jax.ShapeDtypeStruct((B, W, S), jnp.int32),
        interpret=_INTERPRET,
    )(nxT, xyz)
    return jnp.transpose(out, (0, 2, 1))


def _sc_select(words, N, K):
    B, S, W = words.shape
    R = B * S
    info = plsc.get_sparse_core_info()
    NWK = info.num_cores * info.num_subcores
    rows_per = R // NWK
    mesh = plsc.VectorSubcoreMesh(core_axis_name="c", subcore_axis_name="s")

    @functools.partial(
        pl.kernel, mesh=mesh,
        compiler_params=pltpu.CompilerParams(needs_layout_passes=False),
        out_type=jax.ShapeDtypeStruct((R * K,), jnp.int32),
        scratch_types=[
            pltpu.VMEM((rows_per * W,), jnp.int32),
            pltpu.VMEM((W + 16,), jnp.int32),
            pltpu.VMEM((N + 16,), jnp.int32),
            pltpu.VMEM((K,), jnp.int32),
        ],
    )
    def k(words_hbm, out_hbm, wbuf, wlist, idxbuf, obuf):
        wid = lax.axis_index("s") * info.num_cores + lax.axis_index("c")
        base = wid * rows_per
        pltpu.sync_copy(words_hbm.at[pl.ds(base * W, rows_per * W)], wbuf)
        iota = jnp.arange(16, dtype=jnp.int32)

        def row_body(r, _):
            woff = r * W
            ones = jnp.ones((16,), jnp.int32)
            zeros = jnp.zeros((16,), jnp.int32)

            def bc(v):
                return jnp.full((16,), v, jnp.int32)

            def p1(g, nw):
                wv = wbuf[pl.ds(woff + g * 16, 16)]
                m = wv != zeros
                mi = m.astype(jnp.int32)
                pos = plsc.cumsum(mi) + bc(nw) - ones
                plsc.store_scatter(wlist, [pos], bc(g * 16) + iota, mask=m)
                return nw + jnp.sum(mi)

            nw = lax.fori_loop(0, W // 16, p1, jnp.int32(0), unroll=False)

            def p2(j, off):
                wsp = plsc.load_gather(wlist, [bc(j)])
                wval = plsc.load_gather(wbuf, [wsp + bc(woff)])
                m = (lax.shift_right_logical(wval, iota) & ones) != zeros
                mi = m.astype(jnp.int32)
                pos = plsc.cumsum(mi) + bc(off) - ones
                plsc.store_scatter(idxbuf, [pos], wsp * 16 + iota, mask=m)
                return off + jnp.sum(mi)

            lax.fori_loop(0, nw, p2, jnp.int32(0), unroll=False)
            first = plsc.load_gather(idxbuf, [zeros])
            has = (bc(nw) != zeros).astype(jnp.int32)
            first = first * has + jnp.full((16,), N, jnp.int32) * (ones - has)
            for q in range(K // 16):
                obuf[pl.ds(q * 16, 16)] = first
            kfull = jnp.full((16,), K, jnp.int32)

            def p3(j, off):
                wsp = plsc.load_gather(wlist, [bc(j)])
                wval = plsc.load_gather(wbuf, [wsp + bc(woff)])
                m = (lax.shift_right_logical(wval, iota) & ones) != zeros
                mi = m.astype(jnp.int32)
                pos = plsc.cumsum(mi) + bc(off) - ones
                inb = lax.shift_right_logical(pos - kfull, 31) != zeros
                plsc.store_scatter(obuf, [pos], wsp * 16 + iota,
                                   mask=jnp.logical_and(m, inb))
                return off + jnp.sum(mi)

            lax.fori_loop(0, nw, p3, jnp.int32(0), unroll=False)
            pltpu.sync_copy(obuf, out_hbm.at[pl.ds((base + r) * K, K)])
            return 0

        lax.fori_loop(0, rows_per, row_body, 0, unroll=False)

    return k(words.reshape(R * W)).reshape(B, S, K)


def _query_ball_sc(radius, K, xyz, new_xyz):
    words = _ballq_words(xyz, new_xyz, radius)
    return _sc_select(words, xyz.shape[1], K)


def _sq_dist(src, dst):
    return (jnp.sum(src ** 2, -1)[:, :, None]
            + jnp.sum(dst ** 2, -1)[:, None, :]
            - 2.0 * jnp.einsum('bsc,bnc->bsn', src, dst))


def _fps(xyz, npoint):
    B, N, _ = xyz.shape

    def body(i, state):
        centroids, distance, farthest = state
        centroids = centroids.at[:, i].set(farthest)
        centroid = jnp.take_along_axis(xyz, farthest[:, None, None], axis=1)
        dist = jnp.sum((xyz - centroid) ** 2, -1)
        distance = jnp.minimum(distance, dist)
        farthest = jnp.argmax(distance, -1).astype(jnp.int32)
        return centroids, distance, farthest

    centroids = jnp.zeros((B, npoint), dtype=jnp.int32)
    distance = jnp.full((B, N), 1e10, dtype=xyz.dtype)
    farthest = jnp.zeros((B,), dtype=jnp.int32)
    centroids, _, _ = lax.fori_loop(0, npoint, body,
                                    (centroids, distance, farthest))
    return centroids


def _query_ball(radius, nsample, xyz, new_xyz):
    B, N, _ = xyz.shape
    S = new_xyz.shape[1]
    sqrdists = _sq_dist(new_xyz, xyz)
    group_idx = jnp.broadcast_to(jnp.arange(N, dtype=jnp.int32), (B, S, N))
    group_idx = jnp.where(sqrdists > radius ** 2, N, group_idx)
    group_idx = jnp.sort(group_idx, axis=-1)[:, :, :nsample]
    group_first = jnp.broadcast_to(group_idx[:, :, :1], group_idx.shape)
    return jnp.where(group_idx == N, group_first, group_idx)


def _gather_rows(points, idx):
    B = points.shape[0]
    batch = jnp.arange(B).reshape((B,) + (1,) * (idx.ndim - 1))
    return points[batch, idx]


def _sa_layer(cfg, lps, xyz, feats_raw, fstat):
    B, N, _ = xyz.shape
    S, K = cfg['npoint'], cfg['nsample']
    new_xyz = _fps_pallas(xyz, S)
    idx = _query_ball(cfg['radius'], K, xyz, new_xyz)
    gx = _gather_rows(xyz, idx) - new_xyz[:, :, None, :]
    one3 = jnp.ones((3,), jnp.float32)
    zero3 = jnp.zeros((3,), jnp.float32)
    neg3 = jnp.full((3,), _NEG, jnp.float32)
    if feats_raw is None:
        X = gx
        m, sq, g, bt, fl = zero3, one3, one3, zero3, neg3
    else:
        gf = _gather_rows(feats_raw, idx)
        X = jnp.concatenate([gx, gf], axis=-1)
        fm, fsq, fg, fb = fstat
        m = jnp.concatenate([zero3, fm])
        sq = jnp.concatenate([one3, fsq])
        g = jnp.concatenate([one3, fg])
        bt = jnp.concatenate([zero3, fb])
        fl = jnp.concatenate([neg3, jnp.zeros_like(fm)])
    pooled = None
    for i, (W, b, gamma, beta) in enumerate(lps):
        pool = (i == len(lps) - 1)
        z, pooled = _mlp_stage(X, m, sq, g, bt, fl, W, b, pool=pool)
        zm, zsq = _stats(z)
        m, sq, g, bt = zm, zsq, gamma, beta
        fl = jnp.zeros_like(zm)
        X = z
    return new_xyz, pooled, (m, sq, g, bt)


def kernel(input, params):
    xyz = jnp.transpose(input, (0, 2, 1))
    feats, fstat = None, None
    for cfg, lps in zip(_SA_CFGS, params):
        xyz, feats, fstat = _sa_layer(cfg, lps, xyz, feats, fstat)
    f = _epilogue(feats, *fstat)
    return (jnp.transpose(xyz, (0, 2, 1)), jnp.transpose(f, (0, 2, 1)))
